# Initial kernel scaffold; baseline (speedup 1.0000x reference)
import jax
import jax.numpy as jnp
from jax.experimental import pallas as pl


def kernel(x):
    raise NotImplementedError("write your pallas kernel here")


# speedup vs baseline: 2.7342x; 2.7342x over previous
import jax
import jax.numpy as jnp
from jax.experimental import pallas as pl


def _split_body(x_ref, o1_ref, o2_ref):
    o1_ref[...] = x_ref[:, :512]
    o2_ref[...] = x_ref[:, 512:]


def kernel(x):
    B, C, H, W = x.shape
    HW = H * W
    nt = HW // 2
    rows = B * C
    xr = x.reshape(rows, HW)
    blk = 768
    grid = rows // blk
    o1, o2 = pl.pallas_call(
        _split_body,
        grid=(grid,),
        in_specs=[pl.BlockSpec((blk, HW), lambda i: (i, 0))],
        out_specs=[
            pl.BlockSpec((blk, nt), lambda i: (i, 0)),
            pl.BlockSpec((blk, nt), lambda i: (i, 0)),
        ],
        out_shape=[
            jax.ShapeDtypeStruct((rows, nt), x.dtype),
            jax.ShapeDtypeStruct((rows, nt), x.dtype),
        ],
    )(xr)
    X1 = o1.reshape(B, C, H, nt // W)
    X2 = o2.reshape(B, C, H, nt // W)
    return (X1, X2)
